# Initial kernel scaffold; baseline (speedup 1.0000x reference)
import functools

import jax
import jax.numpy as jnp
from jax import lax
from jax.experimental import pallas as pl
from jax.experimental.pallas import tpu as pltpu
import jax.experimental.pallas.tpu_sc as plsc

N_NODES = 10000
NPAD = 10240
NC, NS = 2, 16
NW = NC * NS
RPT = NPAD // NS
K = 128
TC_BLK = 1024


def _sc_agg(dout, ept):
    nchunk = ept // K
    mesh = plsc.VectorSubcoreMesh(core_axis_name="c", subcore_axis_name="s")

    @functools.partial(
        pl.kernel,
        out_type=jax.ShapeDtypeStruct((NC, NPAD, dout), jnp.float32),
        mesh=mesh,
        scratch_types=[
            pltpu.VMEM((K,), jnp.int32),
            pltpu.VMEM((K,), jnp.int32),
            pltpu.VMEM((K, dout), jnp.float32),
            pltpu.VMEM_SHARED((NPAD, dout), jnp.float32),
            pltpu.SemaphoreType.DMA,
        ],
    )
    def agg(hs_hbm, src_hbm, dst_hbm, zrows_hbm, out_hbm, idx_s, idx_d, rows, acc, sem):
        c = lax.axis_index("c")
        s = lax.axis_index("s")
        pltpu.sync_copy(zrows_hbm, acc.at[pl.ds(s * RPT, RPT)])
        plsc.subcore_barrier()

        tile = c * NS + s
        base = tile * ept

        def chunk(k, carry):
            off = base + k * K
            pltpu.sync_copy(src_hbm.at[pl.ds(off, K)], idx_s)
            pltpu.sync_copy(dst_hbm.at[pl.ds(off, K)], idx_d)
            pltpu.async_copy(hs_hbm.at[idx_s], rows, sem).wait()
            pltpu.sync_copy(rows, acc.at[idx_d], add=True)
            return carry

        lax.fori_loop(0, nchunk, chunk, 0)
        plsc.subcore_barrier()
        pltpu.sync_copy(acc.at[pl.ds(s * RPT, RPT)], out_hbm.at[c, pl.ds(s * RPT, RPT)])

    return agg


def _sc_deg(ept):
    nchunk = ept // K
    mesh = plsc.VectorSubcoreMesh(core_axis_name="c", subcore_axis_name="s")

    @functools.partial(
        pl.kernel,
        out_type=jax.ShapeDtypeStruct((NC, NPAD, 16), jnp.float32),
        mesh=mesh,
        scratch_types=[
            pltpu.VMEM((K,), jnp.int32),
            pltpu.VMEM((K, 16), jnp.float32),
            pltpu.VMEM_SHARED((NPAD, 16), jnp.float32),
        ],
    )
    def deg(dst_hbm, ones_hbm, zrows_hbm, out_hbm, idx_d, ones_v, acc):
        c = lax.axis_index("c")
        s = lax.axis_index("s")
        pltpu.sync_copy(zrows_hbm, acc.at[pl.ds(s * RPT, RPT)])
        pltpu.sync_copy(ones_hbm, ones_v)
        plsc.subcore_barrier()

        tile = c * NS + s
        base = tile * ept

        def chunk(k, carry):
            off = base + k * K
            pltpu.sync_copy(dst_hbm.at[pl.ds(off, K)], idx_d)
            pltpu.sync_copy(ones_v, acc.at[idx_d], add=True)
            return carry

        lax.fori_loop(0, nchunk, chunk, 0)
        plsc.subcore_barrier()
        pltpu.sync_copy(acc.at[pl.ds(s * RPT, RPT)], out_hbm.at[c, pl.ds(s * RPT, RPT)])

    return deg


def _tc_first(x_pad, d0, d1, w1):
    grid = (NPAD // TC_BLK,)

    def body(x_ref, d0_ref, d1_ref, w_ref, dis_ref, hs_ref):
        pid = pl.program_id(0)
        deg = (d0_ref[...] + d1_ref[...])[:, 0:1]
        row = lax.broadcasted_iota(jnp.int32, (TC_BLK, 1), 0) + pid * TC_BLK
        valid = (row < N_NODES).astype(jnp.float32)
        dis = valid * lax.rsqrt(jnp.maximum(deg, 1.0))
        dis_ref[...] = dis
        hs_ref[...] = jnp.dot(x_ref[...], w_ref[...],
                              preferred_element_type=jnp.float32) * dis

    return pl.pallas_call(
        body,
        grid=grid,
        in_specs=[
            pl.BlockSpec((TC_BLK, 128), lambda i: (i, 0)),
            pl.BlockSpec((TC_BLK, 16), lambda i: (i, 0)),
            pl.BlockSpec((TC_BLK, 16), lambda i: (i, 0)),
            pl.BlockSpec((128, 128), lambda i: (0, 0)),
        ],
        out_specs=[
            pl.BlockSpec((TC_BLK, 1), lambda i: (i, 0)),
            pl.BlockSpec((TC_BLK, 128), lambda i: (i, 0)),
        ],
        out_shape=[
            jax.ShapeDtypeStruct((NPAD, 1), jnp.float32),
            jax.ShapeDtypeStruct((NPAD, 128), jnp.float32),
        ],
    )(x_pad, d0, d1, w1)


def _tc_mid(a0, a1, dis, b, w):
    din = a0.shape[1]
    dout = w.shape[1]
    grid = (NPAD // TC_BLK,)

    def body(a0_ref, a1_ref, dis_ref, b_ref, w_ref, hs_ref):
        dis_v = dis_ref[...]
        h = jnp.maximum((a0_ref[...] + a1_ref[...]) * dis_v + b_ref[...], 0.0)
        hs_ref[...] = jnp.dot(h, w_ref[...],
                              preferred_element_type=jnp.float32) * dis_v

    return pl.pallas_call(
        body,
        grid=grid,
        in_specs=[
            pl.BlockSpec((TC_BLK, din), lambda i: (i, 0)),
            pl.BlockSpec((TC_BLK, din), lambda i: (i, 0)),
            pl.BlockSpec((TC_BLK, 1), lambda i: (i, 0)),
            pl.BlockSpec((1, din), lambda i: (0, 0)),
            pl.BlockSpec((din, dout), lambda i: (0, 0)),
        ],
        out_specs=pl.BlockSpec((TC_BLK, dout), lambda i: (i, 0)),
        out_shape=jax.ShapeDtypeStruct((NPAD, dout), jnp.float32),
    )(a0, a1, dis, b, w)


def _tc_last(a0, a1, dis, b):
    dout = a0.shape[1]
    grid = (NPAD // TC_BLK,)

    def body(a0_ref, a1_ref, dis_ref, b_ref, out_ref):
        out_ref[...] = (a0_ref[...] + a1_ref[...]) * dis_ref[...] + b_ref[...]

    return pl.pallas_call(
        body,
        grid=grid,
        in_specs=[
            pl.BlockSpec((TC_BLK, dout), lambda i: (i, 0)),
            pl.BlockSpec((TC_BLK, dout), lambda i: (i, 0)),
            pl.BlockSpec((TC_BLK, 1), lambda i: (i, 0)),
            pl.BlockSpec((1, dout), lambda i: (0, 0)),
        ],
        out_specs=pl.BlockSpec((TC_BLK, dout), lambda i: (i, 0)),
        out_shape=jax.ShapeDtypeStruct((NPAD, dout), jnp.float32),
    )(a0, a1, dis, b)


def kernel(x, edge_index, W1, b1, W2, b2, W3, b3, W4, b4, W5, b5):
    n = x.shape[0]
    e = edge_index.shape[1]
    e_tot = e + n
    epad = -(-e_tot // (NW * K)) * (NW * K)
    ept = epad // NW

    ei = edge_index.astype(jnp.int32)
    loop = jnp.arange(n, dtype=jnp.int32)
    pad = jnp.full((epad - e_tot,), N_NODES, dtype=jnp.int32)
    srcp = jnp.concatenate([ei[0], loop, pad])
    dstp = jnp.concatenate([ei[1], loop, pad])

    x_pad = jnp.pad(x, ((0, NPAD - n), (0, 0)))
    ones16 = jnp.ones((K, 16), jnp.float32)
    z16 = jnp.zeros((RPT, 16), jnp.float32)

    degp = _sc_deg(ept)(dstp, ones16, z16)
    dis, hs = _tc_first(x_pad, degp[0], degp[1], W1)

    ws = [W2, W3, W4, W5]
    bs = [b1, b2, b3, b4]
    for i in range(4):
        dout = hs.shape[1]
        zr = jnp.zeros((RPT, dout), jnp.float32)
        aggp = _sc_agg(dout, ept)(hs, srcp, dstp, zr)
        hs = _tc_mid(aggp[0], aggp[1], dis, bs[i].reshape(1, -1), ws[i])

    dout = hs.shape[1]
    zr = jnp.zeros((RPT, dout), jnp.float32)
    aggp = _sc_agg(dout, ept)(hs, srcp, dstp, zr)
    out = _tc_last(aggp[0], aggp[1], dis, b5.reshape(1, -1))
    return out[:n]


# speedup vs baseline: 12.2822x; 12.2822x over previous
import functools

import jax
import jax.numpy as jnp
from jax import lax
from jax.experimental import pallas as pl
from jax.experimental.pallas import tpu as pltpu
import jax.experimental.pallas.tpu_sc as plsc

N_NODES = 10000
NPAD = 10240
NC, NS = 2, 16
NW = NC * NS
RPT = NPAD // NS
K = 128
TC_BLK = 1024


def _sc_agg(dout, ept):
    nchunk = ept // K
    mesh = plsc.VectorSubcoreMesh(core_axis_name="c", subcore_axis_name="s", num_cores=NC, num_subcores=NS)

    @functools.partial(
        pl.kernel,
        out_type=jax.ShapeDtypeStruct((NC, NPAD, dout), jnp.float32),
        mesh=mesh,
        scratch_types=[
            pltpu.VMEM((K,), jnp.int32),
            pltpu.VMEM((K,), jnp.int32),
            pltpu.VMEM((K, dout), jnp.float32),
            pltpu.VMEM_SHARED((NPAD, dout), jnp.float32),
            pltpu.SemaphoreType.DMA,
        ],
        compiler_params=pltpu.CompilerParams(use_tc_tiling_on_sc=False),
    )
    def agg(hs_hbm, src_hbm, dst_hbm, zrows_hbm, out_hbm, idx_s, idx_d, rows, acc, sem):
        c = lax.axis_index("c")
        s = lax.axis_index("s")
        pltpu.sync_copy(zrows_hbm, acc.at[pl.ds(s * RPT, RPT)])
        plsc.subcore_barrier()

        tile = c * NS + s
        base = tile * ept

        def chunk(k, carry):
            off = base + k * K
            pltpu.sync_copy(src_hbm.at[pl.ds(off, K)], idx_s)
            pltpu.sync_copy(dst_hbm.at[pl.ds(off, K)], idx_d)
            pltpu.async_copy(hs_hbm.at[idx_s], rows, sem).wait()
            pltpu.sync_copy(rows, acc.at[idx_d], add=True)
            return carry

        lax.fori_loop(0, nchunk, chunk, 0)
        plsc.subcore_barrier()
        pltpu.sync_copy(acc.at[pl.ds(s * RPT, RPT)], out_hbm.at[c, pl.ds(s * RPT, RPT)])

    return agg


def _sc_deg(ept):
    nchunk = ept // K
    mesh = plsc.VectorSubcoreMesh(core_axis_name="c", subcore_axis_name="s", num_cores=NC, num_subcores=NS)

    @functools.partial(
        pl.kernel,
        out_type=jax.ShapeDtypeStruct((NC, NPAD, 16), jnp.float32),
        mesh=mesh,
        scratch_types=[
            pltpu.VMEM((K,), jnp.int32),
            pltpu.VMEM((K, 16), jnp.float32),
            pltpu.VMEM_SHARED((NPAD, 16), jnp.float32),
        ],
        compiler_params=pltpu.CompilerParams(use_tc_tiling_on_sc=False),
    )
    def deg(dst_hbm, ones_hbm, zrows_hbm, out_hbm, idx_d, ones_v, acc):
        c = lax.axis_index("c")
        s = lax.axis_index("s")
        pltpu.sync_copy(zrows_hbm, acc.at[pl.ds(s * RPT, RPT)])
        pltpu.sync_copy(ones_hbm, ones_v)
        plsc.subcore_barrier()

        tile = c * NS + s
        base = tile * ept

        def chunk(k, carry):
            off = base + k * K
            pltpu.sync_copy(dst_hbm.at[pl.ds(off, K)], idx_d)
            pltpu.sync_copy(ones_v, acc.at[idx_d], add=True)
            return carry

        lax.fori_loop(0, nchunk, chunk, 0)
        plsc.subcore_barrier()
        pltpu.sync_copy(acc.at[pl.ds(s * RPT, RPT)], out_hbm.at[c, pl.ds(s * RPT, RPT)])

    return deg


def _tc_first(x_pad, d0, d1, w1):
    grid = (NPAD // TC_BLK,)

    def body(x_ref, d0_ref, d1_ref, w_ref, dis_ref, hs_ref):
        pid = pl.program_id(0)
        deg = (d0_ref[...] + d1_ref[...])[:, 0:1]
        row = lax.broadcasted_iota(jnp.int32, (TC_BLK, 1), 0) + pid * TC_BLK
        valid = (row < N_NODES).astype(jnp.float32)
        dis = valid * lax.rsqrt(jnp.maximum(deg, 1.0))
        dis_ref[...] = dis
        hs_ref[...] = jnp.dot(x_ref[...], w_ref[...],
                              preferred_element_type=jnp.float32) * dis

    return pl.pallas_call(
        body,
        grid=grid,
        in_specs=[
            pl.BlockSpec((TC_BLK, 128), lambda i: (i, 0)),
            pl.BlockSpec((TC_BLK, 16), lambda i: (i, 0)),
            pl.BlockSpec((TC_BLK, 16), lambda i: (i, 0)),
            pl.BlockSpec((128, 128), lambda i: (0, 0)),
        ],
        out_specs=[
            pl.BlockSpec((TC_BLK, 1), lambda i: (i, 0)),
            pl.BlockSpec((TC_BLK, 128), lambda i: (i, 0)),
        ],
        out_shape=[
            jax.ShapeDtypeStruct((NPAD, 1), jnp.float32),
            jax.ShapeDtypeStruct((NPAD, 128), jnp.float32),
        ],
    )(x_pad, d0, d1, w1)


def _tc_mid(a0, a1, dis, b, w):
    din = a0.shape[1]
    dout = w.shape[1]
    grid = (NPAD // TC_BLK,)

    def body(a0_ref, a1_ref, dis_ref, b_ref, w_ref, hs_ref):
        dis_v = dis_ref[...]
        h = jnp.maximum((a0_ref[...] + a1_ref[...]) * dis_v + b_ref[...], 0.0)
        hs_ref[...] = jnp.dot(h, w_ref[...],
                              preferred_element_type=jnp.float32) * dis_v

    return pl.pallas_call(
        body,
        grid=grid,
        in_specs=[
            pl.BlockSpec((TC_BLK, din), lambda i: (i, 0)),
            pl.BlockSpec((TC_BLK, din), lambda i: (i, 0)),
            pl.BlockSpec((TC_BLK, 1), lambda i: (i, 0)),
            pl.BlockSpec((1, din), lambda i: (0, 0)),
            pl.BlockSpec((din, dout), lambda i: (0, 0)),
        ],
        out_specs=pl.BlockSpec((TC_BLK, dout), lambda i: (i, 0)),
        out_shape=jax.ShapeDtypeStruct((NPAD, dout), jnp.float32),
    )(a0, a1, dis, b, w)


def _tc_last(a0, a1, dis, b):
    dout = a0.shape[1]
    grid = (NPAD // TC_BLK,)

    def body(a0_ref, a1_ref, dis_ref, b_ref, out_ref):
        out_ref[...] = (a0_ref[...] + a1_ref[...]) * dis_ref[...] + b_ref[...]

    return pl.pallas_call(
        body,
        grid=grid,
        in_specs=[
            pl.BlockSpec((TC_BLK, dout), lambda i: (i, 0)),
            pl.BlockSpec((TC_BLK, dout), lambda i: (i, 0)),
            pl.BlockSpec((TC_BLK, 1), lambda i: (i, 0)),
            pl.BlockSpec((1, dout), lambda i: (0, 0)),
        ],
        out_specs=pl.BlockSpec((TC_BLK, dout), lambda i: (i, 0)),
        out_shape=jax.ShapeDtypeStruct((NPAD, dout), jnp.float32),
    )(a0, a1, dis, b)


def kernel(x, edge_index, W1, b1, W2, b2, W3, b3, W4, b4, W5, b5):
    n = x.shape[0]
    e = edge_index.shape[1]
    e_tot = e + n
    epad = -(-e_tot // (NW * K)) * (NW * K)
    ept = epad // NW

    ei = edge_index.astype(jnp.int32)
    loop = jnp.arange(n, dtype=jnp.int32)
    pad = jnp.full((epad - e_tot,), N_NODES, dtype=jnp.int32)
    srcp = jnp.concatenate([ei[0], loop, pad])
    dstp = jnp.concatenate([ei[1], loop, pad])

    x_pad = jnp.pad(x, ((0, NPAD - n), (0, 0)))
    ones16 = jnp.ones((K, 16), jnp.float32)
    z16 = jnp.zeros((RPT, 16), jnp.float32)

    degp = _sc_deg(ept)(dstp, ones16, z16)
    dis, hs = _tc_first(x_pad, degp[0], degp[1], W1)

    ws = [W2, W3, W4, W5]
    bs = [b1, b2, b3, b4]
    for i in range(4):
        dout = hs.shape[1]
        zr = jnp.zeros((RPT, dout), jnp.float32)
        aggp = _sc_agg(dout, ept)(hs, srcp, dstp, zr)
        hs = _tc_mid(aggp[0], aggp[1], dis, bs[i].reshape(1, -1), ws[i])

    dout = hs.shape[1]
    zr = jnp.zeros((RPT, dout), jnp.float32)
    aggp = _sc_agg(dout, ept)(hs, srcp, dstp, zr)
    out = _tc_last(aggp[0], aggp[1], dis, b5.reshape(1, -1))
    return out[:n]


# speedup vs baseline: 13.2982x; 1.0827x over previous
import functools

import jax
import jax.numpy as jnp
from jax import lax
from jax.experimental import pallas as pl
from jax.experimental.pallas import tpu as pltpu
import jax.experimental.pallas.tpu_sc as plsc

N_NODES = 10000
NPAD = 10240
NC, NS = 2, 16
NW = NC * NS
RPT = NPAD // NS
K = 128
NBUF = 2
TC_BLK = 1024


def _sc_agg(dout, ept):
    nchunk = ept // K
    ngroup = nchunk // NBUF
    mesh = plsc.VectorSubcoreMesh(core_axis_name="c", subcore_axis_name="s", num_cores=NC, num_subcores=NS)

    scratch = []
    scratch += [pltpu.VMEM((K,), jnp.int32) for _ in range(NBUF)]
    scratch += [pltpu.VMEM((K,), jnp.int32) for _ in range(NBUF)]
    scratch += [pltpu.VMEM((K, dout), jnp.float32) for _ in range(NBUF)]
    scratch += [pltpu.VMEM_SHARED((NPAD, dout), jnp.float32)]
    scratch += [pltpu.SemaphoreType.DMA for _ in range(NBUF)]

    @functools.partial(
        pl.kernel,
        out_type=jax.ShapeDtypeStruct((NC, NPAD, dout), jnp.float32),
        mesh=mesh,
        scratch_types=scratch,
        compiler_params=pltpu.CompilerParams(use_tc_tiling_on_sc=False),
    )
    def agg(hs_hbm, src_hbm, dst_hbm, zrows_hbm, out_hbm, *scr):
        idx_s = scr[0:NBUF]
        idx_d = scr[NBUF:2 * NBUF]
        rows = scr[2 * NBUF:3 * NBUF]
        acc = scr[3 * NBUF]
        sems = scr[3 * NBUF + 1:3 * NBUF + 1 + NBUF]

        c = lax.axis_index("c")
        s = lax.axis_index("s")
        pltpu.sync_copy(zrows_hbm, acc.at[pl.ds(s * RPT, RPT)])
        plsc.subcore_barrier()

        tile = c * NS + s
        base = tile * ept

        for b in range(NBUF):
            off = base + b * K
            pltpu.sync_copy(src_hbm.at[pl.ds(off, K)], idx_s[b])
            pltpu.sync_copy(dst_hbm.at[pl.ds(off, K)], idx_d[b])
            pltpu.async_copy(hs_hbm.at[idx_s[b]], rows[b], sems[b])

        def group(g, carry):
            for b in range(NBUF):
                pltpu.make_async_copy(hs_hbm.at[idx_s[b]], rows[b], sems[b]).wait()
                pltpu.sync_copy(rows[b], acc.at[idx_d[b]], add=True)
                off = base + ((g + 1) * NBUF + b) * K
                pltpu.sync_copy(src_hbm.at[pl.ds(off, K)], idx_s[b])
                pltpu.sync_copy(dst_hbm.at[pl.ds(off, K)], idx_d[b])
                pltpu.async_copy(hs_hbm.at[idx_s[b]], rows[b], sems[b])
            return carry

        lax.fori_loop(0, ngroup - 1, group, 0)

        for b in range(NBUF):
            pltpu.make_async_copy(hs_hbm.at[idx_s[b]], rows[b], sems[b]).wait()
            pltpu.sync_copy(rows[b], acc.at[idx_d[b]], add=True)

        plsc.subcore_barrier()
        pltpu.sync_copy(acc.at[pl.ds(s * RPT, RPT)], out_hbm.at[c, pl.ds(s * RPT, RPT)])

    return agg


def _sc_deg(ept):
    nchunk = ept // K
    mesh = plsc.VectorSubcoreMesh(core_axis_name="c", subcore_axis_name="s", num_cores=NC, num_subcores=NS)

    @functools.partial(
        pl.kernel,
        out_type=jax.ShapeDtypeStruct((NC, NPAD, 16), jnp.float32),
        mesh=mesh,
        scratch_types=[
            pltpu.VMEM((K,), jnp.int32),
            pltpu.VMEM((K, 16), jnp.float32),
            pltpu.VMEM_SHARED((NPAD, 16), jnp.float32),
        ],
        compiler_params=pltpu.CompilerParams(use_tc_tiling_on_sc=False),
    )
    def deg(dst_hbm, ones_hbm, zrows_hbm, out_hbm, idx_d, ones_v, acc):
        c = lax.axis_index("c")
        s = lax.axis_index("s")
        pltpu.sync_copy(zrows_hbm, acc.at[pl.ds(s * RPT, RPT)])
        pltpu.sync_copy(ones_hbm, ones_v)
        plsc.subcore_barrier()

        tile = c * NS + s
        base = tile * ept

        def chunk(k, carry):
            off = base + k * K
            pltpu.sync_copy(dst_hbm.at[pl.ds(off, K)], idx_d)
            pltpu.sync_copy(ones_v, acc.at[idx_d], add=True)
            return carry

        lax.fori_loop(0, nchunk, chunk, 0)
        plsc.subcore_barrier()
        pltpu.sync_copy(acc.at[pl.ds(s * RPT, RPT)], out_hbm.at[c, pl.ds(s * RPT, RPT)])

    return deg


def _tc_first(x_pad, d0, d1, w1):
    grid = (NPAD // TC_BLK,)

    def body(x_ref, d0_ref, d1_ref, w_ref, dis_ref, hs_ref):
        pid = pl.program_id(0)
        deg = (d0_ref[...] + d1_ref[...])[:, 0:1]
        row = lax.broadcasted_iota(jnp.int32, (TC_BLK, 1), 0) + pid * TC_BLK
        valid = (row < N_NODES).astype(jnp.float32)
        dis = valid * lax.rsqrt(jnp.maximum(deg, 1.0))
        dis_ref[...] = dis
        hs_ref[...] = jnp.dot(x_ref[...], w_ref[...],
                              preferred_element_type=jnp.float32) * dis

    return pl.pallas_call(
        body,
        grid=grid,
        in_specs=[
            pl.BlockSpec((TC_BLK, 128), lambda i: (i, 0)),
            pl.BlockSpec((TC_BLK, 16), lambda i: (i, 0)),
            pl.BlockSpec((TC_BLK, 16), lambda i: (i, 0)),
            pl.BlockSpec((128, 128), lambda i: (0, 0)),
        ],
        out_specs=[
            pl.BlockSpec((TC_BLK, 1), lambda i: (i, 0)),
            pl.BlockSpec((TC_BLK, 128), lambda i: (i, 0)),
        ],
        out_shape=[
            jax.ShapeDtypeStruct((NPAD, 1), jnp.float32),
            jax.ShapeDtypeStruct((NPAD, 128), jnp.float32),
        ],
    )(x_pad, d0, d1, w1)


def _tc_mid(a0, a1, dis, b, w):
    din = a0.shape[1]
    dout = w.shape[1]
    grid = (NPAD // TC_BLK,)

    def body(a0_ref, a1_ref, dis_ref, b_ref, w_ref, hs_ref):
        dis_v = dis_ref[...]
        h = jnp.maximum((a0_ref[...] + a1_ref[...]) * dis_v + b_ref[...], 0.0)
        hs_ref[...] = jnp.dot(h, w_ref[...],
                              preferred_element_type=jnp.float32) * dis_v

    return pl.pallas_call(
        body,
        grid=grid,
        in_specs=[
            pl.BlockSpec((TC_BLK, din), lambda i: (i, 0)),
            pl.BlockSpec((TC_BLK, din), lambda i: (i, 0)),
            pl.BlockSpec((TC_BLK, 1), lambda i: (i, 0)),
            pl.BlockSpec((1, din), lambda i: (0, 0)),
            pl.BlockSpec((din, dout), lambda i: (0, 0)),
        ],
        out_specs=pl.BlockSpec((TC_BLK, dout), lambda i: (i, 0)),
        out_shape=jax.ShapeDtypeStruct((NPAD, dout), jnp.float32),
    )(a0, a1, dis, b, w)


def _tc_last(a0, a1, dis, b):
    dout = a0.shape[1]
    grid = (NPAD // TC_BLK,)

    def body(a0_ref, a1_ref, dis_ref, b_ref, out_ref):
        out_ref[...] = (a0_ref[...] + a1_ref[...]) * dis_ref[...] + b_ref[...]

    return pl.pallas_call(
        body,
        grid=grid,
        in_specs=[
            pl.BlockSpec((TC_BLK, dout), lambda i: (i, 0)),
            pl.BlockSpec((TC_BLK, dout), lambda i: (i, 0)),
            pl.BlockSpec((TC_BLK, 1), lambda i: (i, 0)),
            pl.BlockSpec((1, dout), lambda i: (0, 0)),
        ],
        out_specs=pl.BlockSpec((TC_BLK, dout), lambda i: (i, 0)),
        out_shape=jax.ShapeDtypeStruct((NPAD, dout), jnp.float32),
    )(a0, a1, dis, b)


def kernel(x, edge_index, W1, b1, W2, b2, W3, b3, W4, b4, W5, b5):
    n = x.shape[0]
    e = edge_index.shape[1]
    e_tot = e + n
    epad = -(-e_tot // (NW * K * NBUF)) * (NW * K * NBUF)
    ept = epad // NW

    ei = edge_index.astype(jnp.int32)
    loop = jnp.arange(n, dtype=jnp.int32)
    pad = jnp.full((epad - e_tot,), N_NODES, dtype=jnp.int32)
    srcp = jnp.concatenate([ei[0], loop, pad])
    dstp = jnp.concatenate([ei[1], loop, pad])

    x_pad = jnp.pad(x, ((0, NPAD - n), (0, 0)))
    ones16 = jnp.ones((K, 16), jnp.float32)
    z16 = jnp.zeros((RPT, 16), jnp.float32)

    degp = _sc_deg(ept)(dstp, ones16, z16)
    dis, hs = _tc_first(x_pad, degp[0], degp[1], W1)

    ws = [W2, W3, W4, W5]
    bs = [b1, b2, b3, b4]
    for i in range(4):
        dout = hs.shape[1]
        zr = jnp.zeros((RPT, dout), jnp.float32)
        aggp = _sc_agg(dout, ept)(hs, srcp, dstp, zr)
        hs = _tc_mid(aggp[0], aggp[1], dis, bs[i].reshape(1, -1), ws[i])

    dout = hs.shape[1]
    zr = jnp.zeros((RPT, dout), jnp.float32)
    aggp = _sc_agg(dout, ept)(hs, srcp, dstp, zr)
    out = _tc_last(aggp[0], aggp[1], dis, b5.reshape(1, -1))
    return out[:n]
